# Initial kernel scaffold; baseline (speedup 1.0000x reference)
import jax
import jax.numpy as jnp
from jax.experimental import pallas as pl


def kernel(x, edge_index, W1, b1, W2, b2):
    raise NotImplementedError("write your pallas kernel here")


# speedup vs baseline: 8.2347x; 8.2347x over previous
import functools

import jax
import jax.numpy as jnp
from jax import lax
from jax.experimental import pallas as pl
from jax.experimental.pallas import tpu as pltpu
from jax.experimental.pallas import tpu_sc as plsc

NC = 2
NS = 16
CHUNK = 128

_mesh = plsc.VectorSubcoreMesh(
    core_axis_name="c", subcore_axis_name="s", num_cores=NC, num_subcores=NS
)


def _make_deg_kernel(npad, nch):
    rpt = npad // NS
    half = nch // 2

    @functools.partial(
        pl.kernel,
        out_type=jax.ShapeDtypeStruct((NC, npad), jnp.float32),
        mesh=_mesh,
        scratch_types=[
            pltpu.VMEM((half, CHUNK), jnp.int32),
            pltpu.VMEM((CHUNK,), jnp.float32),
            pltpu.VMEM_SHARED((npad,), jnp.float32),
        ],
    )
    def deg_kernel(dst_hbm, zeros_hbm, out_hbm, idx_v, ones_v, acc_sh):
        c = lax.axis_index("c")
        s = lax.axis_index("s")
        pltpu.sync_copy(
            zeros_hbm.at[pl.ds(s * rpt, rpt)], acc_sh.at[pl.ds(s * rpt, rpt)]
        )

        @pl.loop(0, CHUNK // 16)
        def _(i):
            ones_v[pl.ds(i * 16, 16)] = jnp.full((16,), 1.0, jnp.float32)

        pltpu.sync_copy(dst_hbm.at[s, pl.ds(c * half, half)], idx_v)
        plsc.subcore_barrier()

        @pl.loop(0, half)
        def _(j):
            pltpu.sync_copy(ones_v, acc_sh.at[idx_v.at[j]], add=True)

        plsc.subcore_barrier()
        pltpu.sync_copy(
            acc_sh.at[pl.ds(s * rpt, rpt)], out_hbm.at[c, pl.ds(s * rpt, rpt)]
        )

    return deg_kernel


def _make_msg_kernel(npad, nch, hw):
    rpt = npad // NS

    @functools.partial(
        pl.kernel,
        out_type=jax.ShapeDtypeStruct((NC, npad, hw), jnp.float32),
        mesh=_mesh,
        scratch_types=[
            pltpu.VMEM((nch, CHUNK), jnp.int32),
            pltpu.VMEM((nch, CHUNK), jnp.int32),
            pltpu.VMEM((CHUNK, hw), jnp.float32),
            pltpu.SemaphoreType.DMA,
            pltpu.VMEM_SHARED((npad, hw), jnp.float32),
        ],
    )
    def msg_kernel(g_hbm, src_hbm, dst_hbm, zeros_hbm, out_hbm, srcv, dstv, buf, sem, acc_sh):
        c = lax.axis_index("c")
        s = lax.axis_index("s")
        pltpu.sync_copy(zeros_hbm, acc_sh.at[pl.ds(s * rpt, rpt)])
        pltpu.sync_copy(src_hbm.at[s], srcv)
        pltpu.sync_copy(dst_hbm.at[s], dstv)
        plsc.subcore_barrier()
        gh = g_hbm.at[c]

        @pl.loop(0, nch)
        def _(j):
            pltpu.async_copy(gh.at[srcv.at[j]], buf, sem).wait()
            pltpu.sync_copy(buf, acc_sh.at[dstv.at[j]], add=True)

        plsc.subcore_barrier()
        pltpu.sync_copy(
            acc_sh.at[pl.ds(s * rpt, rpt)], out_hbm.at[c, pl.ds(s * rpt, rpt)]
        )

    return msg_kernel


def _dinv_of(deg_ref):
    deg = jnp.sum(deg_ref[...], axis=1, keepdims=True) + 1.0
    return lax.rsqrt(deg)


def _tc_in_body(deg_ref, x_ref, w_ref, h_ref, g_ref):
    dinv = _dinv_of(deg_ref)
    h = jnp.dot(x_ref[...], w_ref[...], preferred_element_type=jnp.float32)
    h_ref[...] = h
    g = h * dinv
    hw = g.shape[1] // 2
    g_ref[0] = g[:, :hw]
    g_ref[1] = g[:, hw:]


def _tc_mid_body(deg_ref, acc_ref, h1_ref, b_ref, w_ref, h2_ref, g_ref):
    dinv = _dinv_of(deg_ref)
    acc = jnp.concatenate([acc_ref[0], acc_ref[1]], axis=1)
    out1 = jnp.maximum(
        acc * dinv + h1_ref[...] * (dinv * dinv) + b_ref[...], 0.0
    )
    h2 = jnp.dot(out1, w_ref[...], preferred_element_type=jnp.float32)
    h2_ref[...] = h2
    g = h2 * dinv
    hw = g.shape[1] // 2
    g_ref[0] = g[:, :hw]
    g_ref[1] = g[:, hw:]


def _tc_out_body(deg_ref, acc_ref, h2_ref, b_ref, out_ref):
    dinv = _dinv_of(deg_ref)
    acc = jnp.concatenate([acc_ref[0], acc_ref[1]], axis=1)
    out_ref[...] = acc * dinv + h2_ref[...] * (dinv * dinv) + b_ref[...]


def kernel(x, edge_index, W1, b1, W2, b2):
    n, d = x.shape
    e = edge_index.shape[1]
    hw = d // 2

    ept = e // NS
    nch = -(-ept // CHUNK)
    if nch % 2:
        nch += 1
    ept_pad = nch * CHUNK
    blk = 1024
    npad = -(-n // blk) * blk

    src = edge_index[0].reshape(NS, ept)
    dst = edge_index[1].reshape(NS, ept)
    pad = ((0, 0), (0, ept_pad - ept))
    src_r = jnp.pad(src, pad, constant_values=n).reshape(NS, nch, CHUNK)
    dst_r = jnp.pad(dst, pad, constant_values=n).reshape(NS, nch, CHUNK)
    x_pad = jnp.pad(x, ((0, npad - n), (0, 0)))

    zeros1 = jnp.zeros((npad,), jnp.float32)
    zeros2 = jnp.zeros((npad // NS, hw), jnp.float32)

    deg_kernel = _make_deg_kernel(npad, nch)
    msg_kernel = _make_msg_kernel(npad, nch, hw)

    deg2 = deg_kernel(dst_r, zeros1)
    deg_t = deg2.T

    grid = npad // blk
    f32 = jnp.float32
    deg_spec = pl.BlockSpec((blk, NC), lambda i: (i, 0))
    row_spec = pl.BlockSpec((blk, d), lambda i: (i, 0))
    w_spec = pl.BlockSpec((d, d), lambda i: (0, 0))
    b_spec = pl.BlockSpec((1, d), lambda i: (0, 0))
    acc_spec = pl.BlockSpec((NC, blk, hw), lambda i: (0, i, 0))

    tc_in = pl.pallas_call(
        _tc_in_body,
        grid=(grid,),
        in_specs=[deg_spec, row_spec, w_spec],
        out_specs=[row_spec, acc_spec],
        out_shape=[
            jax.ShapeDtypeStruct((npad, d), f32),
            jax.ShapeDtypeStruct((NC, npad, hw), f32),
        ],
    )
    tc_mid = pl.pallas_call(
        _tc_mid_body,
        grid=(grid,),
        in_specs=[deg_spec, acc_spec, row_spec, b_spec, w_spec],
        out_specs=[row_spec, acc_spec],
        out_shape=[
            jax.ShapeDtypeStruct((npad, d), f32),
            jax.ShapeDtypeStruct((NC, npad, hw), f32),
        ],
    )
    tc_out = pl.pallas_call(
        _tc_out_body,
        grid=(grid,),
        in_specs=[deg_spec, acc_spec, row_spec, b_spec],
        out_specs=row_spec,
        out_shape=jax.ShapeDtypeStruct((npad, d), f32),
    )

    h1, g1 = tc_in(deg_t, x_pad, W1)
    acc1 = msg_kernel(g1, src_r, dst_r, zeros2)
    h2, g2 = tc_mid(deg_t, acc1, h1, b1.reshape(1, d), W2)
    acc2 = msg_kernel(g2, src_r, dst_r, zeros2)
    out = tc_out(deg_t, acc2, h2, b2.reshape(1, d))
    return out[:n]


# speedup vs baseline: 9.0751x; 1.1021x over previous
import functools

import jax
import jax.numpy as jnp
from jax import lax
from jax.experimental import pallas as pl
from jax.experimental.pallas import tpu as pltpu
from jax.experimental.pallas import tpu_sc as plsc

NC = 2
NS = 16
CHUNK = 128

_mesh = plsc.VectorSubcoreMesh(
    core_axis_name="c", subcore_axis_name="s", num_cores=NC, num_subcores=NS
)


def _make_deg_kernel(npad, nch):
    rpt = npad // NS
    half = nch // 2

    @functools.partial(
        pl.kernel,
        out_type=jax.ShapeDtypeStruct((NC, npad), jnp.float32),
        mesh=_mesh,
        scratch_types=[
            pltpu.VMEM((half, CHUNK), jnp.int32),
            pltpu.VMEM((CHUNK,), jnp.float32),
            pltpu.VMEM_SHARED((npad,), jnp.float32),
        ],
    )
    def deg_kernel(dst_hbm, zeros_hbm, out_hbm, idx_v, ones_v, acc_sh):
        c = lax.axis_index("c")
        s = lax.axis_index("s")
        pltpu.sync_copy(
            zeros_hbm.at[pl.ds(s * rpt, rpt)], acc_sh.at[pl.ds(s * rpt, rpt)]
        )

        @pl.loop(0, CHUNK // 16)
        def _(i):
            ones_v[pl.ds(i * 16, 16)] = jnp.full((16,), 1.0, jnp.float32)

        pltpu.sync_copy(dst_hbm.at[s, pl.ds(c * half, half)], idx_v)
        plsc.subcore_barrier()

        @pl.loop(0, half)
        def _(j):
            pltpu.sync_copy(ones_v, acc_sh.at[idx_v.at[j]], add=True)

        plsc.subcore_barrier()
        pltpu.sync_copy(
            acc_sh.at[pl.ds(s * rpt, rpt)], out_hbm.at[c, pl.ds(s * rpt, rpt)]
        )

    return deg_kernel


def _make_msg_kernel(npad, nch, hw):
    rpt = npad // NS
    grp = next((g for g in (16, 8) if nch % g == 0), nch)
    ngrp = nch // grp

    @functools.partial(
        pl.kernel,
        out_type=jax.ShapeDtypeStruct((NC, npad, hw), jnp.float32),
        mesh=_mesh,
        scratch_types=[
            pltpu.VMEM((grp, CHUNK), jnp.int32),
            pltpu.VMEM((grp, CHUNK), jnp.int32),
            pltpu.VMEM((2, CHUNK, hw), jnp.float32),
            pltpu.SemaphoreType.DMA,
            pltpu.VMEM_SHARED((npad, hw), jnp.float32),
        ],
    )
    def msg_kernel(g_hbm, src_hbm, dst_hbm, zeros_hbm, out_hbm, srcv, dstv, buf, sem, acc_sh):
        c = lax.axis_index("c")
        s = lax.axis_index("s")
        pltpu.sync_copy(zeros_hbm, acc_sh.at[pl.ds(s * rpt, rpt)])
        plsc.subcore_barrier()
        gh = g_hbm.at[c]

        for g in range(ngrp):
            pltpu.sync_copy(src_hbm.at[s, pl.ds(g * grp, grp)], srcv)
            pltpu.sync_copy(dst_hbm.at[s, pl.ds(g * grp, grp)], dstv)
            pltpu.make_async_copy(gh.at[srcv.at[0]], buf.at[0], sem).start()

            @pl.loop(0, grp, step=2)
            def _(i):
                for b in range(2):
                    j = i + b
                    pltpu.make_async_copy(gh.at[srcv.at[j]], buf.at[b], sem).wait()
                    nxt = j + 1

                    @pl.when(nxt < grp)
                    def _():
                        pltpu.make_async_copy(
                            gh.at[srcv.at[nxt]], buf.at[1 - b], sem
                        ).start()

                    pltpu.sync_copy(buf.at[b], acc_sh.at[dstv.at[j]], add=True)

        plsc.subcore_barrier()
        pltpu.sync_copy(
            acc_sh.at[pl.ds(s * rpt, rpt)], out_hbm.at[c, pl.ds(s * rpt, rpt)]
        )

    return msg_kernel


def _dinv_of(deg_ref):
    deg = jnp.sum(deg_ref[...], axis=1, keepdims=True) + 1.0
    return lax.rsqrt(deg)


def _tc_in_body(deg_ref, x_ref, w_ref, h_ref, g_ref):
    dinv = _dinv_of(deg_ref)
    h = jnp.dot(x_ref[...], w_ref[...], preferred_element_type=jnp.float32)
    h_ref[...] = h
    g = h * dinv
    hw = g.shape[1] // 2
    g_ref[0] = g[:, :hw]
    g_ref[1] = g[:, hw:]


def _tc_mid_body(deg_ref, acc_ref, h1_ref, b_ref, w_ref, h2_ref, g_ref):
    dinv = _dinv_of(deg_ref)
    acc = jnp.concatenate([acc_ref[0], acc_ref[1]], axis=1)
    out1 = jnp.maximum(
        acc * dinv + h1_ref[...] * (dinv * dinv) + b_ref[...], 0.0
    )
    h2 = jnp.dot(out1, w_ref[...], preferred_element_type=jnp.float32)
    h2_ref[...] = h2
    g = h2 * dinv
    hw = g.shape[1] // 2
    g_ref[0] = g[:, :hw]
    g_ref[1] = g[:, hw:]


def _tc_out_body(deg_ref, acc_ref, h2_ref, b_ref, out_ref):
    dinv = _dinv_of(deg_ref)
    acc = jnp.concatenate([acc_ref[0], acc_ref[1]], axis=1)
    out_ref[...] = acc * dinv + h2_ref[...] * (dinv * dinv) + b_ref[...]


def kernel(x, edge_index, W1, b1, W2, b2):
    n, d = x.shape
    e = edge_index.shape[1]
    hw = d // 2

    ept = e // NS
    nch = -(-ept // CHUNK)
    if nch % 2:
        nch += 1
    ept_pad = nch * CHUNK
    blk = 1024
    npad = -(-n // blk) * blk

    src = edge_index[0].reshape(NS, ept)
    dst = edge_index[1].reshape(NS, ept)
    pad = ((0, 0), (0, ept_pad - ept))
    src_r = jnp.pad(src, pad, constant_values=n).reshape(NS, nch, CHUNK)
    dst_r = jnp.pad(dst, pad, constant_values=n).reshape(NS, nch, CHUNK)
    x_pad = jnp.pad(x, ((0, npad - n), (0, 0)))

    zeros1 = jnp.zeros((npad,), jnp.float32)
    zeros2 = jnp.zeros((npad // NS, hw), jnp.float32)

    deg_kernel = _make_deg_kernel(npad, nch)
    msg_kernel = _make_msg_kernel(npad, nch, hw)

    deg2 = deg_kernel(dst_r, zeros1)
    deg_t = deg2.T

    grid = npad // blk
    f32 = jnp.float32
    deg_spec = pl.BlockSpec((blk, NC), lambda i: (i, 0))
    row_spec = pl.BlockSpec((blk, d), lambda i: (i, 0))
    w_spec = pl.BlockSpec((d, d), lambda i: (0, 0))
    b_spec = pl.BlockSpec((1, d), lambda i: (0, 0))
    acc_spec = pl.BlockSpec((NC, blk, hw), lambda i: (0, i, 0))

    tc_in = pl.pallas_call(
        _tc_in_body,
        grid=(grid,),
        in_specs=[deg_spec, row_spec, w_spec],
        out_specs=[row_spec, acc_spec],
        out_shape=[
            jax.ShapeDtypeStruct((npad, d), f32),
            jax.ShapeDtypeStruct((NC, npad, hw), f32),
        ],
    )
    tc_mid = pl.pallas_call(
        _tc_mid_body,
        grid=(grid,),
        in_specs=[deg_spec, acc_spec, row_spec, b_spec, w_spec],
        out_specs=[row_spec, acc_spec],
        out_shape=[
            jax.ShapeDtypeStruct((npad, d), f32),
            jax.ShapeDtypeStruct((NC, npad, hw), f32),
        ],
    )
    tc_out = pl.pallas_call(
        _tc_out_body,
        grid=(grid,),
        in_specs=[deg_spec, acc_spec, row_spec, b_spec],
        out_specs=row_spec,
        out_shape=jax.ShapeDtypeStruct((npad, d), f32),
    )

    h1, g1 = tc_in(deg_t, x_pad, W1)
    acc1 = msg_kernel(g1, src_r, dst_r, zeros2)
    h2, g2 = tc_mid(deg_t, acc1, h1, b1.reshape(1, d), W2)
    acc2 = msg_kernel(g2, src_r, dst_r, zeros2)
    out = tc_out(deg_t, acc2, h2, b2.reshape(1, d))
    return out[:n]


# speedup vs baseline: 9.3283x; 1.0279x over previous
import functools

import jax
import jax.numpy as jnp
from jax import lax
from jax.experimental import pallas as pl
from jax.experimental.pallas import tpu as pltpu
from jax.experimental.pallas import tpu_sc as plsc

NC = 2
NS = 16
CHUNK = 128

_mesh = plsc.VectorSubcoreMesh(
    core_axis_name="c", subcore_axis_name="s", num_cores=NC, num_subcores=NS
)


def _make_deg_kernel(npad, nch):
    rpt = npad // NS
    half = nch // 2

    @functools.partial(
        pl.kernel,
        out_type=jax.ShapeDtypeStruct((NC, npad), jnp.float32),
        mesh=_mesh,
        scratch_types=[
            pltpu.VMEM((half, CHUNK), jnp.int32),
            pltpu.VMEM((CHUNK,), jnp.float32),
            pltpu.VMEM_SHARED((npad,), jnp.float32),
        ],
    )
    def deg_kernel(dst_hbm, zeros_hbm, out_hbm, idx_v, ones_v, acc_sh):
        c = lax.axis_index("c")
        s = lax.axis_index("s")
        pltpu.sync_copy(
            zeros_hbm.at[pl.ds(s * rpt, rpt)], acc_sh.at[pl.ds(s * rpt, rpt)]
        )

        @pl.loop(0, CHUNK // 16)
        def _(i):
            ones_v[pl.ds(i * 16, 16)] = jnp.full((16,), 1.0, jnp.float32)

        pltpu.sync_copy(dst_hbm.at[s, pl.ds(c * half, half)], idx_v)
        plsc.subcore_barrier()

        @pl.loop(0, half)
        def _(j):
            pltpu.sync_copy(ones_v, acc_sh.at[idx_v.at[j]], add=True)

        plsc.subcore_barrier()
        pltpu.sync_copy(
            acc_sh.at[pl.ds(s * rpt, rpt)], out_hbm.at[c, pl.ds(s * rpt, rpt)]
        )

    return deg_kernel


def _make_msg_kernel(npad, nch, hw):
    rpt = npad // NS
    grp = next((g for g in (16, 8) if nch % g == 0), nch)
    ngrp = nch // grp

    @functools.partial(
        pl.kernel,
        out_type=jax.ShapeDtypeStruct((NC, npad, hw), jnp.float32),
        mesh=_mesh,
        scratch_types=[
            pltpu.VMEM((grp, CHUNK), jnp.int32),
            pltpu.VMEM((grp, CHUNK), jnp.int32),
            pltpu.VMEM((2, CHUNK, hw), jnp.float32),
            pltpu.SemaphoreType.DMA,
            pltpu.SemaphoreType.DMA,
            pltpu.VMEM_SHARED((npad, hw), jnp.float32),
        ],
    )
    def msg_kernel(
        g_hbm, src_hbm, dst_hbm, out_hbm, srcv, dstv, buf, gsem, ssem, acc_sh
    ):
        c = lax.axis_index("c")
        s = lax.axis_index("s")
        gh = g_hbm.at[c]
        pltpu.sync_copy(gh.at[pl.ds(s * rpt, rpt)], acc_sh.at[pl.ds(s * rpt, rpt)])
        plsc.subcore_barrier()

        def drain_scatter():
            pltpu.make_async_copy(buf.at[0], acc_sh.at[dstv.at[0]], ssem).wait()

        for g in range(ngrp):
            pltpu.sync_copy(src_hbm.at[s, pl.ds(g * grp, grp)], srcv)
            pltpu.sync_copy(dst_hbm.at[s, pl.ds(g * grp, grp)], dstv)
            pltpu.make_async_copy(gh.at[srcv.at[0]], buf.at[0], gsem).start()

            @pl.loop(0, grp, step=2)
            def _(i, g=g):
                for b in range(2):
                    j = i + b
                    pltpu.make_async_copy(gh.at[srcv.at[j]], buf.at[b], gsem).wait()
                    if g == 0:

                        @pl.when(j > 0)
                        def _():
                            drain_scatter()

                    else:
                        drain_scatter()
                    nxt = j + 1

                    @pl.when(nxt < grp)
                    def _():
                        pltpu.make_async_copy(
                            gh.at[srcv.at[nxt]], buf.at[1 - b], gsem
                        ).start()

                    pltpu.make_async_copy(
                        buf.at[b], acc_sh.at[dstv.at[j]], ssem
                    ).start(add=True)

        drain_scatter()

        plsc.subcore_barrier()
        pltpu.sync_copy(
            acc_sh.at[pl.ds(s * rpt, rpt)], out_hbm.at[c, pl.ds(s * rpt, rpt)]
        )

    return msg_kernel


def _dinv_of(deg_ref):
    deg = jnp.sum(deg_ref[...], axis=1, keepdims=True) + 1.0
    return lax.rsqrt(deg)


def _tc_in_body(deg_ref, x_ref, w_ref, g_ref):
    dinv = _dinv_of(deg_ref)
    g = jnp.dot(x_ref[...], w_ref[...], preferred_element_type=jnp.float32) * dinv
    hw = g.shape[1] // 2
    g_ref[0] = g[:, :hw]
    g_ref[1] = g[:, hw:]


def _tc_mid_body(deg_ref, acc_ref, b_ref, w_ref, g_ref):
    dinv = _dinv_of(deg_ref)
    acc = jnp.concatenate([acc_ref[0], acc_ref[1]], axis=1)
    out1 = jnp.maximum(acc * dinv + b_ref[...], 0.0)
    g = jnp.dot(out1, w_ref[...], preferred_element_type=jnp.float32) * dinv
    hw = g.shape[1] // 2
    g_ref[0] = g[:, :hw]
    g_ref[1] = g[:, hw:]


def _tc_out_body(deg_ref, acc_ref, b_ref, out_ref):
    dinv = _dinv_of(deg_ref)
    acc = jnp.concatenate([acc_ref[0], acc_ref[1]], axis=1)
    out_ref[...] = acc * dinv + b_ref[...]


def kernel(x, edge_index, W1, b1, W2, b2):
    n, d = x.shape
    e = edge_index.shape[1]
    hw = d // 2

    ept = e // NS
    nch = -(-ept // CHUNK)
    if nch % 2:
        nch += 1
    ept_pad = nch * CHUNK
    blk = 1024
    npad = -(-n // blk) * blk

    src = edge_index[0].reshape(NS, ept)
    dst = edge_index[1].reshape(NS, ept)
    pad = ((0, 0), (0, ept_pad - ept))
    src_r = jnp.pad(src, pad, constant_values=n).reshape(NS, nch, CHUNK)
    dst_r = jnp.pad(dst, pad, constant_values=n).reshape(NS, nch, CHUNK)
    x_pad = jnp.pad(x, ((0, npad - n), (0, 0)))

    zeros1 = jnp.zeros((npad,), jnp.float32)

    deg_kernel = _make_deg_kernel(npad, nch)
    msg_kernel = _make_msg_kernel(npad, nch, hw)

    deg2 = deg_kernel(dst_r, zeros1)
    deg_t = deg2.T

    grid = npad // blk
    f32 = jnp.float32
    deg_spec = pl.BlockSpec((blk, NC), lambda i: (i, 0))
    row_spec = pl.BlockSpec((blk, d), lambda i: (i, 0))
    w_spec = pl.BlockSpec((d, d), lambda i: (0, 0))
    b_spec = pl.BlockSpec((1, d), lambda i: (0, 0))
    acc_spec = pl.BlockSpec((NC, blk, hw), lambda i: (0, i, 0))

    tc_in = pl.pallas_call(
        _tc_in_body,
        grid=(grid,),
        in_specs=[deg_spec, row_spec, w_spec],
        out_specs=acc_spec,
        out_shape=jax.ShapeDtypeStruct((NC, npad, hw), f32),
    )
    tc_mid = pl.pallas_call(
        _tc_mid_body,
        grid=(grid,),
        in_specs=[deg_spec, acc_spec, b_spec, w_spec],
        out_specs=acc_spec,
        out_shape=jax.ShapeDtypeStruct((NC, npad, hw), f32),
    )
    tc_out = pl.pallas_call(
        _tc_out_body,
        grid=(grid,),
        in_specs=[deg_spec, acc_spec, b_spec],
        out_specs=row_spec,
        out_shape=jax.ShapeDtypeStruct((npad, d), f32),
    )

    g1 = tc_in(deg_t, x_pad, W1)
    acc1 = msg_kernel(g1, src_r, dst_r)
    g2 = tc_mid(deg_t, acc1, b1.reshape(1, d), W2)
    acc2 = msg_kernel(g2, src_r, dst_r)
    out = tc_out(deg_t, acc2, b2.reshape(1, d))
    return out[:n]
